# baseline (speedup 1.0000x reference)
import functools

import jax
import jax.numpy as jnp
from jax import lax
from jax.experimental import pallas as pl
from jax.experimental.pallas import tpu as pltpu
from jax.experimental.pallas import tpu_sc as plsc

N = 10000
D = 128
E = 320000

NC = 2
NS = 16
HC = D // NC
CHUNK = 128
NCHUNK = 160
E_PAD = NS * NCHUNK * CHUNK
N_PAD = 10240
ROWS_PER_TILE = N_PAD // NS
BR = 512
GRID = N_PAD // BR


def _sc_segment_sum(with_deg):
    mesh = plsc.VectorSubcoreMesh(core_axis_name="c", subcore_axis_name="s")

    out_type = [jax.ShapeDtypeStruct((NC, N_PAD, HC), jnp.float32)]
    scratch = [
        pltpu.VMEM((NCHUNK, CHUNK), jnp.int32),
        pltpu.VMEM((NCHUNK, CHUNK), jnp.int32),
        pltpu.VMEM((CHUNK, HC), jnp.float32),
        pltpu.VMEM((CHUNK, HC), jnp.float32),
        pltpu.VMEM((CHUNK, HC), jnp.float32),
        pltpu.VMEM((CHUNK, HC), jnp.float32),
        pltpu.VMEM_SHARED((N_PAD, HC), jnp.float32),
    ] + [pltpu.SemaphoreType.DMA] * 8
    if with_deg:
        out_type.append(jax.ShapeDtypeStruct((NC, NS, N_PAD), jnp.float32))
        scratch.append(pltpu.VMEM((N_PAD,), jnp.float32))

    @functools.partial(
        pl.kernel, out_type=out_type, mesh=mesh, scratch_types=scratch,
        compiler_params=pltpu.CompilerParams(
            needs_layout_passes=False, use_tc_tiling_on_sc=False))
    def seg(table_hbm, srcs_hbm, dsts_hbm, zeros_hbm, out_hbm, *rest):
        if with_deg:
            deg_hbm = rest[0]
            rest = rest[1:]
        src_v, dst_v = rest[0], rest[1]
        bufs = rest[2:6]
        acc = rest[6]
        gsems = rest[7:11]
        ssems = rest[11:15]
        if with_deg:
            deg_v = rest[15]
        c = lax.axis_index("c")
        s = lax.axis_index("s")
        r0 = s * ROWS_PER_TILE

        pltpu.sync_copy(srcs_hbm.at[pl.ds((c * NS + s) * NCHUNK, NCHUNK)], src_v)
        pltpu.sync_copy(dsts_hbm.at[pl.ds(s * NCHUNK, NCHUNK)], dst_v)
        pltpu.sync_copy(zeros_hbm.at[pl.ds(r0, ROWS_PER_TILE)],
                        acc.at[pl.ds(r0, ROWS_PER_TILE)])
        if with_deg:
            def zbody(i, carry):
                deg_v[pl.ds(i * 16, 16)] = jnp.zeros((16,), jnp.float32)
                return carry
            lax.fori_loop(0, N_PAD // 16, zbody, 0)
        plsc.subcore_barrier()

        ones16 = jnp.ones((16,), jnp.float32)

        def chunk_deg(j):
            if with_deg:
                @pl.when((j < NCHUNK // 2) == (c == 0))
                def _():
                    for v in range(CHUNK // 16):
                        idx16 = dst_v[j, pl.ds(v * 16, 16)]
                        plsc.addupdate_scatter(deg_v, [idx16], ones16)

        def fire_gather(j, b):
            pltpu.async_copy(table_hbm.at[src_v.at[j]], bufs[b], gsems[b])

        def wait_gather(j, b):
            pltpu.make_async_copy(table_hbm.at[src_v.at[j]], bufs[b], gsems[b]).wait()

        def fire_scatter(j, b):
            pltpu.async_copy(bufs[b], acc.at[dst_v.at[j]], ssems[b], add=True)

        def wait_scatter(j, b):
            pltpu.make_async_copy(bufs[b], acc.at[dst_v.at[j]], ssems[b]).wait()

        NBUF, LAG = 4, 2
        for b in range(LAG):
            fire_gather(b, b)
        for j in range(LAG):
            fire_gather(j + LAG, j + LAG)
            wait_gather(j, j)
            fire_scatter(j, j)
            chunk_deg(j)

        def steady(g, carry):
            j0 = LAG + g * NBUF
            for k in range(NBUF):
                j = j0 + k
                b = (LAG + k) % NBUF
                bn = k
                wait_scatter(j - LAG, bn)
                fire_gather(j + LAG, bn)
                wait_gather(j, b)
                fire_scatter(j, b)
                chunk_deg(j)
            return carry

        lax.fori_loop(0, (NCHUNK - 2 * LAG) // NBUF, steady, 0)

        for k in range(LAG):
            j = NCHUNK - LAG + k
            b = (LAG + k) % NBUF
            wait_gather(j, b)
            fire_scatter(j, b)
            chunk_deg(j)
        for b in range(NBUF):
            wait_scatter(NCHUNK - NBUF + b, b)

        if with_deg:
            pltpu.sync_copy(deg_v, deg_hbm.at[c, s])
        plsc.subcore_barrier()
        pltpu.sync_copy(acc.at[pl.ds(r0, ROWS_PER_TILE)],
                        out_hbm.at[c, pl.ds(r0, ROWS_PER_TILE)])

    return seg


def _tc1_body(x_ref, wl_ref, wr_ref, t1_ref, xr_ref):
    x = x_ref[...]
    xw = lax.dot_general(x, wl_ref[...], (((1,), (1,)), ((), ())))
    t1_ref[0] = xw[:, :HC]
    t1_ref[1] = xw[:, HC:]
    xr_ref[...] = lax.dot_general(x, wr_ref[...], (((1,), (1,)), ((), ())))


def _tc2_body(p_ref, dp_ref, xr1_ref, bl1_ref, bl2_ref, wl2_ref, wr2_ref,
              t2_ref, xr2b_ref, rdeg_ref):
    s = jnp.concatenate([p_ref[0], p_ref[1]], axis=1)
    dpt = jnp.transpose(dp_ref[...])
    deg = jnp.maximum(jnp.sum(dpt, axis=1, keepdims=True), 1.0)
    rdeg = 1.0 / deg
    h = jnp.maximum(s * rdeg + bl1_ref[...] + xr1_ref[...], 0.0)
    t2 = lax.dot_general(h, wl2_ref[...], (((1,), (1,)), ((), ())))
    t2_ref[0] = t2[:, :HC]
    t2_ref[1] = t2[:, HC:]
    xr2b_ref[...] = lax.dot_general(h, wr2_ref[...], (((1,), (1,)), ((), ()))) + bl2_ref[...]
    rdeg_ref[...] = jnp.broadcast_to(rdeg, (BR, D))


def _tc3_body(q_ref, rdeg_ref, xr2b_ref, out_ref):
    s = jnp.concatenate([q_ref[0], q_ref[1]], axis=1)
    out_ref[...] = s * rdeg_ref[...] + xr2b_ref[...]


def _row_spec(cols):
    return pl.BlockSpec((BR, cols), lambda i: (i, 0))


def _full_spec(r, c):
    return pl.BlockSpec((r, c), lambda i: (0, 0))


def _part_spec():
    return pl.BlockSpec((NC, BR, HC), lambda i: (0, i, 0))


@jax.jit
def kernel(x, edge_index, Wl1, bl1, Wr1, Wl2, bl2, Wr2):
    src = edge_index[0]
    dst = edge_index[1]
    pad = E_PAD - E
    src_p = jnp.concatenate([src, jnp.zeros((pad,), jnp.int32)]).reshape(NS, NCHUNK, CHUNK)
    src_r = jnp.stack([src_p, src_p + N_PAD]).reshape(NC * NS * NCHUNK, CHUNK)
    dst_r = jnp.concatenate([dst, jnp.full((pad,), N, jnp.int32)]).reshape(
        NS * NCHUNK, CHUNK)
    zeros_acc = jnp.zeros((N_PAD, HC), jnp.float32)
    x_pad = jnp.pad(x, ((0, N_PAD - N), (0, 0)))

    t1, xr1 = pl.pallas_call(
        _tc1_body,
        grid=(GRID,),
        in_specs=[_row_spec(D), _full_spec(D, D), _full_spec(D, D)],
        out_specs=[_part_spec(), _row_spec(D)],
        out_shape=[jax.ShapeDtypeStruct((NC, N_PAD, HC), jnp.float32),
                   jax.ShapeDtypeStruct((N_PAD, D), jnp.float32)],
    )(x_pad, Wl1, Wr1)

    part1, degpart = _sc_segment_sum(True)(
        t1.reshape(NC * N_PAD, HC), src_r, dst_r, zeros_acc)

    t2, xr2b, rdeg = pl.pallas_call(
        _tc2_body,
        grid=(GRID,),
        in_specs=[_part_spec(), pl.BlockSpec((NC * NS, BR), lambda i: (0, i)),
                  _row_spec(D), _full_spec(1, D), _full_spec(1, D),
                  _full_spec(D, D), _full_spec(D, D)],
        out_specs=[_part_spec(), _row_spec(D), _row_spec(D)],
        out_shape=[jax.ShapeDtypeStruct((NC, N_PAD, HC), jnp.float32),
                   jax.ShapeDtypeStruct((N_PAD, D), jnp.float32),
                   jax.ShapeDtypeStruct((N_PAD, D), jnp.float32)],
    )(part1, degpart.reshape(NC * NS, N_PAD), xr1, bl1.reshape(1, D),
      bl2.reshape(1, D), Wl2, Wr2)

    (part2,) = _sc_segment_sum(False)(
        t2.reshape(NC * N_PAD, HC), src_r, dst_r, zeros_acc)

    out = pl.pallas_call(
        _tc3_body,
        grid=(GRID,),
        in_specs=[_part_spec(), _row_spec(D), _row_spec(D)],
        out_specs=_row_spec(D),
        out_shape=jax.ShapeDtypeStruct((N_PAD, D), jnp.float32),
    )(part2, rdeg, xr2b)
    return out[:N]


# speedup vs baseline: 1.0099x; 1.0099x over previous
import functools

import jax
import jax.numpy as jnp
from jax import lax
from jax.experimental import pallas as pl
from jax.experimental.pallas import tpu as pltpu
from jax.experimental.pallas import tpu_sc as plsc

N = 10000
D = 128
E = 320000

NC = 2
NS = 16
HC = D // NC
CHUNK = 128
NCHUNK = 160
E_PAD = NS * NCHUNK * CHUNK
N_PAD = 10240
ROWS_PER_TILE = N_PAD // NS
BR = 512
GRID = N_PAD // BR


def _sc_segment_sum(with_deg):
    mesh = plsc.VectorSubcoreMesh(core_axis_name="c", subcore_axis_name="s")

    out_type = [jax.ShapeDtypeStruct((NC, N_PAD, HC), jnp.float32)]
    scratch = [
        pltpu.VMEM((NCHUNK, CHUNK), jnp.int32),
        pltpu.VMEM((NCHUNK, CHUNK), jnp.int32),
        pltpu.VMEM((CHUNK, HC), jnp.float32),
        pltpu.VMEM((CHUNK, HC), jnp.float32),
        pltpu.VMEM((CHUNK, HC), jnp.float32),
        pltpu.VMEM((CHUNK, HC), jnp.float32),
        pltpu.VMEM_SHARED((N_PAD, HC), jnp.float32),
    ] + [pltpu.SemaphoreType.DMA] * 8
    if with_deg:
        out_type.append(jax.ShapeDtypeStruct((NC, NS, N_PAD), jnp.float32))
        scratch.append(pltpu.VMEM((N_PAD,), jnp.float32))

    @functools.partial(
        pl.kernel, out_type=out_type, mesh=mesh, scratch_types=scratch,
        compiler_params=pltpu.CompilerParams(
            needs_layout_passes=False, use_tc_tiling_on_sc=False))
    def seg(table_hbm, srcs_hbm, dsts_hbm, zeros_hbm, out_hbm, *rest):
        if with_deg:
            deg_hbm = rest[0]
            rest = rest[1:]
        src_v, dst_v = rest[0], rest[1]
        bufs = rest[2:6]
        acc = rest[6]
        gsems = rest[7:11]
        ssems = rest[11:15]
        if with_deg:
            deg_v = rest[15]
        c = lax.axis_index("c")
        s = lax.axis_index("s")
        r0 = s * ROWS_PER_TILE

        pltpu.sync_copy(srcs_hbm.at[pl.ds((c * NS + s) * NCHUNK, NCHUNK)], src_v)
        pltpu.sync_copy(dsts_hbm.at[pl.ds(s * NCHUNK, NCHUNK)], dst_v)
        pltpu.sync_copy(zeros_hbm.at[pl.ds(r0, ROWS_PER_TILE)],
                        acc.at[pl.ds(r0, ROWS_PER_TILE)])
        if with_deg:
            def zbody(i, carry):
                deg_v[pl.ds(i * 16, 16)] = jnp.zeros((16,), jnp.float32)
                return carry
            lax.fori_loop(0, N_PAD // 16, zbody, 0)
        plsc.subcore_barrier()

        ones16 = jnp.ones((16,), jnp.float32)

        def chunk_deg(j):
            if with_deg:
                @pl.when((j < NCHUNK // 2) == (c == 0))
                def _():
                    for v in range(CHUNK // 16):
                        idx16 = dst_v[j, pl.ds(v * 16, 16)]
                        plsc.addupdate_scatter(deg_v, [idx16], ones16)

        def fire_gather(j, b):
            pltpu.async_copy(table_hbm.at[src_v.at[j]], bufs[b], gsems[b])

        def wait_gather(j, b):
            pltpu.make_async_copy(table_hbm.at[src_v.at[j]], bufs[b], gsems[b]).wait()

        def fire_scatter(j, b):
            pltpu.async_copy(bufs[b], acc.at[dst_v.at[j]], ssems[b], add=True)

        def wait_scatter(j, b):
            pltpu.make_async_copy(bufs[b], acc.at[dst_v.at[j]], ssems[b]).wait()

        NBUF, LAG = 4, 2
        for b in range(LAG):
            fire_gather(b, b)
        for j in range(LAG):
            fire_gather(j + LAG, j + LAG)
            wait_gather(j, j)
            fire_scatter(j, j)
            chunk_deg(j)

        def steady(g, carry):
            j0 = LAG + g * NBUF
            for k in range(NBUF):
                j = j0 + k
                b = (LAG + k) % NBUF
                bn = k
                wait_scatter(j - LAG, bn)
                fire_gather(j + LAG, bn)
                wait_gather(j, b)
                fire_scatter(j, b)
                chunk_deg(j)
            return carry

        lax.fori_loop(0, (NCHUNK - 2 * LAG) // NBUF, steady, 0)

        for k in range(LAG):
            j = NCHUNK - LAG + k
            b = (LAG + k) % NBUF
            wait_gather(j, b)
            fire_scatter(j, b)
            chunk_deg(j)
        for b in range(NBUF):
            wait_scatter(NCHUNK - NBUF + b, b)

        if with_deg:
            pltpu.sync_copy(deg_v, deg_hbm.at[c, s])
        plsc.subcore_barrier()
        pltpu.sync_copy(acc.at[pl.ds(r0, ROWS_PER_TILE)],
                        out_hbm.at[c, pl.ds(r0, ROWS_PER_TILE)])

    return seg


def _tc1_body(x_ref, wl_ref, wr_ref, t1_ref, xr_ref):
    x = x_ref[...]
    xw = lax.dot_general(x, wl_ref[...], (((1,), (1,)), ((), ())))
    t1_ref[0] = xw[:, :HC]
    t1_ref[1] = xw[:, HC:]
    xr_ref[...] = lax.dot_general(x, wr_ref[...], (((1,), (1,)), ((), ())))


def _rdeg(dp):
    dpt = jnp.transpose(dp)
    deg = jnp.maximum(jnp.sum(dpt, axis=1, keepdims=True), 1.0)
    return 1.0 / deg


def _tc2_body(p_ref, dp_ref, xr1_ref, bl1_ref, bl2_ref, wl2_ref, wr2_ref,
              t2_ref, xr2b_ref):
    s = jnp.concatenate([p_ref[0], p_ref[1]], axis=1)
    h = jnp.maximum(s * _rdeg(dp_ref[...]) + bl1_ref[...] + xr1_ref[...], 0.0)
    t2 = lax.dot_general(h, wl2_ref[...], (((1,), (1,)), ((), ())))
    t2_ref[0] = t2[:, :HC]
    t2_ref[1] = t2[:, HC:]
    xr2b_ref[...] = lax.dot_general(h, wr2_ref[...], (((1,), (1,)), ((), ()))) + bl2_ref[...]


def _tc3_body(q_ref, dp_ref, xr2b_ref, out_ref):
    s = jnp.concatenate([q_ref[0], q_ref[1]], axis=1)
    out_ref[...] = s * _rdeg(dp_ref[...]) + xr2b_ref[...]


def _row_spec(cols):
    return pl.BlockSpec((BR, cols), lambda i: (i, 0))


def _full_spec(r, c):
    return pl.BlockSpec((r, c), lambda i: (0, 0))


def _part_spec():
    return pl.BlockSpec((NC, BR, HC), lambda i: (0, i, 0))


@jax.jit
def kernel(x, edge_index, Wl1, bl1, Wr1, Wl2, bl2, Wr2):
    src = edge_index[0]
    dst = edge_index[1]
    pad = E_PAD - E
    src_p = jnp.concatenate([src, jnp.zeros((pad,), jnp.int32)]).reshape(NS, NCHUNK, CHUNK)
    src_r = jnp.stack([src_p, src_p + N_PAD]).reshape(NC * NS * NCHUNK, CHUNK)
    dst_r = jnp.concatenate([dst, jnp.full((pad,), N, jnp.int32)]).reshape(
        NS * NCHUNK, CHUNK)
    zeros_acc = jnp.zeros((N_PAD, HC), jnp.float32)
    x_pad = jnp.pad(x, ((0, N_PAD - N), (0, 0)))

    t1, xr1 = pl.pallas_call(
        _tc1_body,
        grid=(GRID,),
        in_specs=[_row_spec(D), _full_spec(D, D), _full_spec(D, D)],
        out_specs=[_part_spec(), _row_spec(D)],
        out_shape=[jax.ShapeDtypeStruct((NC, N_PAD, HC), jnp.float32),
                   jax.ShapeDtypeStruct((N_PAD, D), jnp.float32)],
    )(x_pad, Wl1, Wr1)

    part1, degpart = _sc_segment_sum(True)(
        t1.reshape(NC * N_PAD, HC), src_r, dst_r, zeros_acc)

    dp = degpart.reshape(NC * NS, N_PAD)
    dp_spec = pl.BlockSpec((NC * NS, BR), lambda i: (0, i))

    t2, xr2b = pl.pallas_call(
        _tc2_body,
        grid=(GRID,),
        in_specs=[_part_spec(), dp_spec,
                  _row_spec(D), _full_spec(1, D), _full_spec(1, D),
                  _full_spec(D, D), _full_spec(D, D)],
        out_specs=[_part_spec(), _row_spec(D)],
        out_shape=[jax.ShapeDtypeStruct((NC, N_PAD, HC), jnp.float32),
                   jax.ShapeDtypeStruct((N_PAD, D), jnp.float32)],
    )(part1, dp, xr1, bl1.reshape(1, D), bl2.reshape(1, D), Wl2, Wr2)

    (part2,) = _sc_segment_sum(False)(
        t2.reshape(NC * N_PAD, HC), src_r, dst_r, zeros_acc)

    out = pl.pallas_call(
        _tc3_body,
        grid=(GRID,),
        in_specs=[_part_spec(), dp_spec, _row_spec(D)],
        out_specs=_row_spec(D),
        out_shape=jax.ShapeDtypeStruct((N_PAD, D), jnp.float32),
    )(part2, dp, xr2b)
    return out[:N]
